# baseline (speedup 1.0000x reference)
import functools
import math

import jax
import jax.numpy as jnp
from jax import lax
from jax.experimental import pallas as pl

B, J, O, M = 128, 40, 25, 64
D = 256
SCALE = 100.0
JO = J * O
P = 64


def _fused_kernel(pt_ref, no_ref, tjr_ref, jd_ref, tmr_ref, rem_ref,
                  wops_ref, wma_ref, ops_ref, ma_ref, edge_ref):
    f32 = jnp.float32
    pt = pt_ref[0].reshape(JO, M)
    edge_ref[0] = pt * (1.0 / SCALE)

    avg = jnp.sum(pt, axis=1, keepdims=True) * (1.0 / (M * SCALE))
    pos_mask = (pt > 0.0).astype(f32)
    nelig = jnp.sum(pos_mask, axis=1, keepdims=True) * (1.0 / M)

    tjr = tjr_ref[0]
    jd = jd_ref[0]
    sched = jnp.where(jd > 0.0, 0.0, tjr - jnp.min(tjr))
    no = no_ref[0]

    r = lax.broadcasted_iota(jnp.int32, (JO, 1), 0)
    jrow = r // O
    o_row = (r - jrow * O).astype(f32)
    j1h = (lax.broadcasted_iota(jnp.int32, (JO, J), 1) == jrow).astype(f32)
    nr = jnp.dot(j1h, jnp.concatenate([no, sched], axis=1),
                 preferred_element_type=f32)
    no_row = nr[:, 0:1]
    sched_row = nr[:, 1:2]
    opready = jnp.where(o_row == no_row, sched_row, 0.0) * (1.0 / SCALE)

    p_i = lax.broadcasted_iota(jnp.int32, (P, D), 0).astype(f32)
    d_i = lax.broadcasted_iota(jnp.int32, (P, D), 1)
    d_par = (d_i & 1).astype(f32)
    d_even = (d_i - (d_i & 1)).astype(f32)
    ang = p_i * jnp.exp(d_even * (-math.log(10000.0) / D)) + d_par * (math.pi / 2.0)
    pe_tab = jnp.sin(ang)

    pos_row = o_row + no_row
    p1h = (pos_row == lax.broadcasted_iota(jnp.int32, (JO, P), 1).astype(f32)).astype(f32)

    cols = jnp.concatenate([avg, nelig, opready, p1h], axis=1)
    wbig = jnp.concatenate([wops_ref[...], pe_tab], axis=0)
    ops_ref[0] = jnp.dot(cols, wbig,
                         preferred_element_type=f32).reshape(J, O, D)

    tmr = tmr_ref[0]
    a_ma = (tmr - jnp.min(tmr)) * (1.0 / SCALE)
    nem = jnp.sum(pos_mask, axis=0, keepdims=True).reshape(M, 1)
    rem = jnp.sum(rem_ref[0])
    frac = nem * (1.0 / (rem + 1e-6))
    ma_ref[0] = jnp.dot(jnp.concatenate([a_ma, frac], axis=1), wma_ref[...],
                        preferred_element_type=f32)


@functools.partial(jax.jit, static_argnames=())
def kernel(proc_times, next_op, time_job_ready, job_done, time_ma_ready,
           pad_mask, op_scheduled, W_ops, W_ma):
    f32 = jnp.float32
    no_col = next_op.astype(f32).reshape(B, J, 1)
    tjr_col = time_job_ready.reshape(B, J, 1)
    jd_col = job_done.astype(f32).reshape(B, J, 1)
    tmr_col = time_ma_ready.reshape(B, M, 1)
    rem_col = jnp.logical_not(jnp.logical_or(pad_mask, op_scheduled)) \
        .astype(f32).reshape(B, JO, 1)
    wopsT = W_ops.T
    wmaT = W_ma.T

    bspec = lambda shape: pl.BlockSpec((1,) + shape, lambda b: (b, 0, 0))
    wspec = lambda shape: pl.BlockSpec(shape, lambda b: (0, 0))

    ops, ma, edge = pl.pallas_call(
        _fused_kernel,
        grid=(B,),
        in_specs=[
            pl.BlockSpec((1, J, O, M), lambda b: (b, 0, 0, 0)),
            bspec((J, 1)),
            bspec((J, 1)),
            bspec((J, 1)),
            bspec((M, 1)),
            bspec((JO, 1)),
            wspec((3, D)),
            wspec((2, D)),
        ],
        out_specs=[
            pl.BlockSpec((1, J, O, D), lambda b: (b, 0, 0, 0)),
            bspec((M, D)),
            bspec((JO, M)),
        ],
        out_shape=[
            jax.ShapeDtypeStruct((B, J, O, D), f32),
            jax.ShapeDtypeStruct((B, M, D), f32),
            jax.ShapeDtypeStruct((B, JO, M), f32),
        ],
    )(proc_times, no_col, tjr_col, jd_col, tmr_col, rem_col, wopsT, wmaT)

    return ops, ma, edge


# speedup vs baseline: 1.3176x; 1.3176x over previous
import functools
import math

import jax
import jax.numpy as jnp
from jax import lax
from jax.experimental import pallas as pl

B, J, O, M = 128, 40, 25, 64
D = 256
SCALE = 100.0
JO = J * O
OP = 32
JOP = J * OP
P = 64


def _fused_kernel(pt_ref, no_ref, tjr_ref, jd_ref, tmr_ref, rem_ref,
                  wops_ref, wma_ref, ops_ref, ma_ref, edge_ref):
    f32 = jnp.float32
    b = pl.program_id(0)
    pt = pt_ref[0]
    edge_ref[0] = pt.reshape(JO, M) * (1.0 / SCALE)

    pt2 = jnp.concatenate(
        [pt, jnp.zeros((J, OP - O, M), f32)], axis=1).reshape(JOP, M)
    avg = jnp.sum(pt2, axis=1, keepdims=True) * (1.0 / (M * SCALE))
    pos_mask = (pt2 > 0.0).astype(f32)
    nelig = jnp.sum(pos_mask, axis=1, keepdims=True) * (1.0 / M)

    no_row = no_ref[pl.ds(b, 1), :]
    tjr = tjr_ref[pl.ds(b, 1), :]
    jd = jd_ref[pl.ds(b, 1), :]
    sched = jnp.where(jd > 0.0, 0.0, tjr - jnp.min(tjr))

    r = lax.broadcasted_iota(jnp.int32, (JOP, 1), 0)
    jrow = r >> 5
    o_row = (r & (OP - 1)).astype(f32)
    j1h = (lax.broadcasted_iota(jnp.int32, (JOP, J), 1) == jrow).astype(f32)
    ns = jnp.concatenate([no_row, sched], axis=0)
    nr = lax.dot_general(j1h, ns, (((1,), (1,)), ((), ())),
                         preferred_element_type=f32)
    no_r = nr[:, 0:1]
    sched_r = nr[:, 1:2]
    opready = jnp.where(o_row == no_r, sched_r, 0.0) * (1.0 / SCALE)

    p_i = lax.broadcasted_iota(jnp.int32, (P, D), 0).astype(f32)
    d_i = lax.broadcasted_iota(jnp.int32, (P, D), 1)
    d_par = (d_i & 1).astype(f32)
    d_even = (d_i - (d_i & 1)).astype(f32)
    ang = p_i * jnp.exp(d_even * (-math.log(10000.0) / D)) + d_par * (math.pi / 2.0)
    pe_tab = jnp.sin(ang)

    pos = o_row + no_r
    p1h = (pos == lax.broadcasted_iota(jnp.int32, (JOP, P), 1).astype(f32)
           ).astype(f32)

    cols = jnp.concatenate([avg, nelig, opready, p1h], axis=1)
    wbig = jnp.concatenate([wops_ref[...], pe_tab], axis=0)
    ops32 = jnp.dot(cols, wbig, preferred_element_type=f32)
    ops_ref[0] = ops32.reshape(J, OP, D)[:, :O, :]

    tmr = tmr_ref[pl.ds(b, 1), :]
    a_ma = (tmr - jnp.min(tmr)) * (1.0 / SCALE)
    nem = jnp.sum(pos_mask, axis=0, keepdims=True)
    rem = jnp.sum(rem_ref[pl.ds(b, 1), :])
    frac = nem * (1.0 / (rem + 1e-6))
    mam = jnp.concatenate([a_ma, frac], axis=0)
    ma_ref[0] = lax.dot_general(mam, wma_ref[...], (((0,), (0,)), ((), ())),
                                preferred_element_type=f32)


@functools.partial(jax.jit, static_argnames=())
def kernel(proc_times, next_op, time_job_ready, job_done, time_ma_ready,
           pad_mask, op_scheduled, W_ops, W_ma):
    f32 = jnp.float32
    no_f = next_op.astype(f32)
    jd_f = job_done.astype(f32)
    rem_f = jnp.logical_not(jnp.logical_or(pad_mask, op_scheduled)) \
        .astype(f32).reshape(B, JO)
    wopsT = W_ops.T
    wmaT = W_ma.T

    full = lambda shape: pl.BlockSpec(shape, lambda b: (0,) * len(shape))

    ops, ma, edge = pl.pallas_call(
        _fused_kernel,
        grid=(B,),
        in_specs=[
            pl.BlockSpec((1, J, O, M), lambda b: (b, 0, 0, 0)),
            full((B, J)),
            full((B, J)),
            full((B, J)),
            full((B, M)),
            full((B, JO)),
            full((3, D)),
            full((2, D)),
        ],
        out_specs=[
            pl.BlockSpec((1, J, O, D), lambda b: (b, 0, 0, 0)),
            pl.BlockSpec((1, M, D), lambda b: (b, 0, 0)),
            pl.BlockSpec((1, JO, M), lambda b: (b, 0, 0)),
        ],
        out_shape=[
            jax.ShapeDtypeStruct((B, J, O, D), f32),
            jax.ShapeDtypeStruct((B, M, D), f32),
            jax.ShapeDtypeStruct((B, JO, M), f32),
        ],
    )(proc_times, no_f, time_job_ready, jd_f, time_ma_ready, rem_f, wopsT, wmaT)

    return ops, ma, edge


# speedup vs baseline: 1.4588x; 1.1072x over previous
import functools
import math

import jax
import jax.numpy as jnp
from jax import lax
from jax.experimental import pallas as pl

B, J, O, M = 128, 40, 25, 64
D = 256
SCALE = 100.0
JO = J * O
OP = 32
JOP = J * OP
P = 64
NB = 4
R = NB * JOP


def _fused_kernel(pt_ref, no_ref, tjr_ref, jd_ref, tmr_ref, rem_ref,
                  wops_ref, wma_ref, ops_ref, ma_ref, edge_ref):
    f32 = jnp.float32
    b = pl.program_id(0)
    pt = pt_ref[...]
    edge_ref[...] = pt.reshape(NB, JO, M) * (1.0 / SCALE)

    pt2 = jnp.concatenate(
        [pt, jnp.zeros((NB, J, OP - O, M), f32)], axis=2).reshape(R, M)
    avg = jnp.sum(pt2, axis=1, keepdims=True) * (1.0 / (M * SCALE))
    pos_mask = (pt2 > 0.0).astype(f32)
    nelig = jnp.sum(pos_mask, axis=1, keepdims=True) * (1.0 / M)

    r = lax.broadcasted_iota(jnp.int32, (R, 1), 0)
    o_row = (r & (OP - 1)).astype(f32)
    r1 = lax.broadcasted_iota(jnp.int32, (JOP, 1), 0)
    j1h = (lax.broadcasted_iota(jnp.int32, (JOP, J), 1) == (r1 >> 5)) \
        .astype(f32)
    nr_parts = []
    for n in range(NB):
        no_n = no_ref[pl.ds(b * NB + n, 1), :]
        tjr_n = tjr_ref[pl.ds(b * NB + n, 1), :]
        jd_n = jd_ref[pl.ds(b * NB + n, 1), :]
        sched_n = jnp.where(jd_n > 0.0, 0.0, tjr_n - jnp.min(tjr_n))
        nr_parts.append(lax.dot_general(
            j1h, jnp.concatenate([no_n, sched_n], axis=0),
            (((1,), (1,)), ((), ())), preferred_element_type=f32))
    nr = jnp.concatenate(nr_parts, axis=0)
    no_r = nr[:, 0:1]
    sched_r = nr[:, 1:2]
    opready = jnp.where(o_row == no_r, sched_r, 0.0) * (1.0 / SCALE)

    p_i = lax.broadcasted_iota(jnp.int32, (P, D), 0).astype(f32)
    d_i = lax.broadcasted_iota(jnp.int32, (P, D), 1)
    d_par = (d_i & 1).astype(f32)
    d_even = (d_i - (d_i & 1)).astype(f32)
    ang = p_i * jnp.exp(d_even * (-math.log(10000.0) / D)) + d_par * (math.pi / 2.0)
    pe_tab = jnp.sin(ang)

    pos = o_row + no_r
    p1h = (pos == lax.broadcasted_iota(jnp.int32, (R, P), 1).astype(f32)
           ).astype(f32)

    cols = jnp.concatenate([avg, nelig, opready, p1h], axis=1)
    wbig = jnp.concatenate([wops_ref[...], pe_tab], axis=0)
    ops32 = jnp.dot(cols, wbig, preferred_element_type=f32)
    ops_ref[...] = ops32.reshape(NB * J, OP, D)[:, :O, :] \
        .reshape(NB, J, O, D)

    nem = jnp.sum(pos_mask.reshape(NB, JOP, M), axis=1)
    for n in range(NB):
        tmr_n = tmr_ref[pl.ds(b * NB + n, 1), :]
        a_ma_n = (tmr_n - jnp.min(tmr_n)) * (1.0 / SCALE)
        rem_n = jnp.sum(rem_ref[pl.ds(b * NB + n, 1), :])
        frac_n = nem[n:n + 1, :] * (1.0 / (rem_n + 1e-6))
        mam = jnp.concatenate([a_ma_n, frac_n], axis=0)
        ma_ref[n] = lax.dot_general(mam, wma_ref[...],
                                    (((0,), (0,)), ((), ())),
                                    preferred_element_type=f32)


@functools.partial(jax.jit, static_argnames=())
def kernel(proc_times, next_op, time_job_ready, job_done, time_ma_ready,
           pad_mask, op_scheduled, W_ops, W_ma):
    f32 = jnp.float32
    no_f = next_op.astype(f32)
    jd_f = job_done.astype(f32)
    rem_f = jnp.logical_not(jnp.logical_or(pad_mask, op_scheduled)) \
        .astype(f32).reshape(B, JO)
    wopsT = W_ops.T
    wmaT = W_ma.T

    full = lambda shape: pl.BlockSpec(shape, lambda b: (0,) * len(shape))

    ops, ma, edge = pl.pallas_call(
        _fused_kernel,
        grid=(B // NB,),
        in_specs=[
            pl.BlockSpec((NB, J, O, M), lambda b: (b, 0, 0, 0)),
            full((B, J)),
            full((B, J)),
            full((B, J)),
            full((B, M)),
            full((B, JO)),
            full((3, D)),
            full((2, D)),
        ],
        out_specs=[
            pl.BlockSpec((NB, J, O, D), lambda b: (b, 0, 0, 0)),
            pl.BlockSpec((NB, M, D), lambda b: (b, 0, 0)),
            pl.BlockSpec((NB, JO, M), lambda b: (b, 0, 0)),
        ],
        out_shape=[
            jax.ShapeDtypeStruct((B, J, O, D), f32),
            jax.ShapeDtypeStruct((B, M, D), f32),
            jax.ShapeDtypeStruct((B, JO, M), f32),
        ],
    )(proc_times, no_f, time_job_ready, jd_f, time_ma_ready, rem_f, wopsT, wmaT)

    return ops, ma, edge


# speedup vs baseline: 1.8081x; 1.2394x over previous
import functools
import math

import jax
import jax.numpy as jnp
from jax import lax
from jax.experimental import pallas as pl
from jax.experimental.pallas import tpu as pltpu

B, J, O, M = 128, 40, 25, 64
D = 256
SCALE = 100.0
JO = J * O
OP = 32
JOP = J * OP
P = 64
C_AVG, C_NEL, C_RDY = 61, 62, 63
NB = 4
R = NB * JOP


def _fused_kernel(pt_ref, no_ref, tjr_ref, jd_ref, tmr_ref, rem_ref,
                  wops_ref, wma_ref, ops_ref, ma_ref, tab_ref, j1h_ref):
    f32 = jnp.float32
    b = pl.program_id(0)

    @pl.when(b == 0)
    def _init():
        p_i = lax.broadcasted_iota(jnp.int32, (P, D), 0)
        d_i = lax.broadcasted_iota(jnp.int32, (P, D), 1)
        d_par = (d_i & 1).astype(f32)
        d_even = (d_i - (d_i & 1)).astype(f32)
        ang = (p_i.astype(f32) * jnp.exp(d_even * (-math.log(10000.0) / D))
               + d_par * (math.pi / 2.0))
        pe = jnp.sin(ang)
        w = wops_ref[...]
        pe = jnp.where(p_i == C_AVG, w[0:1, :], pe)
        pe = jnp.where(p_i == C_NEL, w[1:2, :], pe)
        pe = jnp.where(p_i == C_RDY, w[2:3, :], pe)
        tab_ref[...] = pe
        r1 = lax.broadcasted_iota(jnp.int32, (JOP, 1), 0)
        j1h_ref[...] = (lax.broadcasted_iota(jnp.int32, (JOP, J), 1)
                        == (r1 >> 5)).astype(f32)

    pt = pt_ref[...]

    pt2 = jnp.concatenate(
        [pt, jnp.zeros((NB, J, OP - O, M), f32)], axis=2).reshape(R, M)
    pos_mask = (pt2 > 0.0).astype(f32)
    ones_avg = jnp.full((M, 1), 1.0 / (M * SCALE), f32)
    ones_nel = jnp.full((M, 1), 1.0 / M, f32)
    avg = jnp.dot(pt2, ones_avg, preferred_element_type=f32)
    nelig = jnp.dot(pos_mask, ones_nel, preferred_element_type=f32)

    r = lax.broadcasted_iota(jnp.int32, (R, 1), 0)
    o_row = (r & (OP - 1)).astype(f32)
    j1h = j1h_ref[...]
    nr_parts = []
    for n in range(NB):
        no_n = no_ref[pl.ds(b * NB + n, 1), :]
        tjr_n = tjr_ref[pl.ds(b * NB + n, 1), :]
        jd_n = jd_ref[pl.ds(b * NB + n, 1), :]
        sched_n = jnp.where(jd_n > 0.0, 0.0, tjr_n - jnp.min(tjr_n))
        nr_parts.append(lax.dot_general(
            j1h, jnp.concatenate([no_n, sched_n], axis=0),
            (((1,), (1,)), ((), ())), preferred_element_type=f32))
    nr = jnp.concatenate(nr_parts, axis=0)
    no_r = nr[:, 0:1]
    sched_r = nr[:, 1:2]
    opready = jnp.where(o_row == no_r, sched_r, 0.0) * (1.0 / SCALE)

    pos = o_row + no_r
    l_i = lax.broadcasted_iota(jnp.int32, (R, P), 1)
    g = jnp.where(l_i == C_AVG, avg,
                  jnp.where(l_i == C_NEL, nelig,
                            jnp.where(l_i == C_RDY, opready,
                                      (pos == l_i.astype(f32)).astype(f32))))
    ops32 = jnp.dot(g, tab_ref[...], preferred_element_type=f32)
    ops_ref[...] = ops32.reshape(NB * J, OP, D)[:, :O, :] \
        .reshape(NB, J, O, D)

    nem = jnp.sum(pos_mask.reshape(NB, JOP, M), axis=1)
    for n in range(NB):
        tmr_n = tmr_ref[pl.ds(b * NB + n, 1), :]
        a_ma_n = (tmr_n - jnp.min(tmr_n)) * (1.0 / SCALE)
        rem_n = jnp.sum(rem_ref[pl.ds(b * NB + n, 1), :])
        frac_n = nem[n:n + 1, :] * (1.0 / (rem_n + 1e-6))
        mam = jnp.concatenate([a_ma_n, frac_n], axis=0)
        ma_ref[n] = lax.dot_general(mam, wma_ref[...],
                                    (((0,), (0,)), ((), ())),
                                    preferred_element_type=f32)


@functools.partial(jax.jit, static_argnames=())
def kernel(proc_times, next_op, time_job_ready, job_done, time_ma_ready,
           pad_mask, op_scheduled, W_ops, W_ma):
    f32 = jnp.float32
    no_f = next_op.astype(f32)
    jd_f = job_done.astype(f32)
    rem_f = jnp.logical_not(jnp.logical_or(pad_mask, op_scheduled)) \
        .astype(f32).reshape(B, JO)
    wopsT = W_ops.T
    wmaT = W_ma.T

    full = lambda shape: pl.BlockSpec(shape, lambda b: (0,) * len(shape))

    ops, ma = pl.pallas_call(
        _fused_kernel,
        grid=(B // NB,),
        in_specs=[
            pl.BlockSpec((NB, J, O, M), lambda b: (b, 0, 0, 0)),
            full((B, J)),
            full((B, J)),
            full((B, J)),
            full((B, M)),
            full((B, JO)),
            full((3, D)),
            full((2, D)),
        ],
        out_specs=[
            pl.BlockSpec((NB, J, O, D), lambda b: (b, 0, 0, 0)),
            pl.BlockSpec((NB, M, D), lambda b: (b, 0, 0)),
        ],
        out_shape=[
            jax.ShapeDtypeStruct((B, J, O, D), f32),
            jax.ShapeDtypeStruct((B, M, D), f32),
        ],
        scratch_shapes=[
            pltpu.VMEM((P, D), f32),
            pltpu.VMEM((JOP, J), f32),
        ],
    )(proc_times, no_f, time_job_ready, jd_f, time_ma_ready, rem_f, wopsT, wmaT)

    edge = (proc_times * (1.0 / SCALE)).reshape(B, JO, M)
    return ops, ma, edge
